# baseline (speedup 1.0000x reference)
import jax
import jax.numpy as jnp
from jax import lax
from jax.experimental import pallas as pl
from jax.experimental.pallas import tpu as pltpu
from jax.experimental.pallas import tpu_sc as plsc

_NC = 2
_NS = 16
_NW = _NC * _NS
_L = 16


def _seg_pos_sc(n):
    bpw = 8
    blk = (n // (_NW * bpw)) // _L * _L
    tail = n - _NW * bpw * blk
    assert blk > 0 and tail % _L == 0 and tail < blk
    h = _L
    chunk = bpw * blk
    tbase = _NW * chunk

    mesh = plsc.VectorSubcoreMesh(core_axis_name="c", subcore_axis_name="s",
                                  num_cores=_NC, num_subcores=_NS)

    def body(ids_hbm, mp_hbm, out_hbm, in_v0, in_v1, out_v0, out_v1, mp_v,
             in_sem0, in_sem1, out_sem0, out_sem1):
        in_bufs = (in_v0, in_v1)
        out_bufs = (out_v0, out_v1)
        in_sems = (in_sem0, in_sem1)
        out_sems = (out_sem0, out_sem1)
        wid = lax.axis_index("s") * _NC + lax.axis_index("c")
        base = wid * chunk
        iota = lax.iota(jnp.int32, _L)

        pltpu.sync_copy(mp_hbm, mp_v)
        mp = mp_v[...]

        def in_desc(gbase, rows, b):
            return pltpu.make_async_copy(
                ids_hbm.at[pl.ds(gbase - h, rows + h)],
                in_bufs[b].at[pl.ds(0, rows + h)], in_sems[b])

        def in_desc0(b):
            return pltpu.make_async_copy(
                ids_hbm.at[pl.ds(0, blk)],
                in_bufs[b].at[pl.ds(h, blk)], in_sems[b])

        def start_in(k, b, start=True):
            if k == 0:
                @pl.when(wid == 0)
                def _():
                    d = in_desc0(b)
                    d.start() if start else d.wait()

                @pl.when(wid > 0)
                def _():
                    d = in_desc(base, blk, b)
                    d.start() if start else d.wait()
            else:
                d = in_desc(base + k * blk, blk, b)
                d.start() if start else d.wait()

        def out_desc(gbase, rows, b):
            return pltpu.make_async_copy(
                out_bufs[b].at[pl.ds(0, 4 * rows)],
                out_hbm.at[pl.ds(4 * gbase, 4 * rows)], out_sems[b])

        def compute(b, nv, gbase):
            inb = in_bufs[b]
            outb = out_bufs[b]

            @plsc.parallel_loop(0, nv, 1, unroll=4)
            def _(i):
                e = i * _L
                cur = inb[pl.ds(e + h, _L)]
                prev = plsc.load_gather(inb, [iota + (e + h - 1)])
                gi = iota + (gbase + e)
                d0 = jnp.where((cur != prev) | (gi == 0), 1, 0).astype(jnp.int32)
                d1 = jnp.where(cur == 0, 1, 0).astype(jnp.int32)
                d2 = jnp.where((cur > 0) & (cur < mp), 1, 0).astype(jnp.int32)
                d3 = jnp.where(cur == mp, 1, 0).astype(jnp.int32)
                oi = 4 * iota + (4 * e)
                plsc.store_scatter(outb, [oi], d0)
                plsc.store_scatter(outb, [oi + 1], d1)
                plsc.store_scatter(outb, [oi + 2], d2)
                plsc.store_scatter(outb, [oi + 3], d3)

        last = _NW - 1
        tail_in = pltpu.make_async_copy(
            ids_hbm.at[pl.ds(tbase - h, tail + h)],
            in_bufs[0].at[pl.ds(0, tail + h)], in_sems[0])
        tail_out = pltpu.make_async_copy(
            out_bufs[0].at[pl.ds(0, 4 * tail)],
            out_hbm.at[pl.ds(4 * tbase, 4 * tail)], out_sems[0])

        start_in(0, 0)
        start_in(1, 1)
        for k in range(bpw):
            b = k & 1
            start_in(k, b, start=False)
            if k >= 2:
                out_desc(base + (k - 2) * blk, blk, b).wait()
            compute(b, blk // _L, base + k * blk)
            out_desc(base + k * blk, blk, b).start()
            if k + 2 < bpw:
                start_in(k + 2, b)
            elif k + 2 == bpw and tail:
                @pl.when(wid == last)
                def _():
                    tail_in.start()

        if tail:
            @pl.when(wid == last)
            def _():
                tail_in.wait()
                out_desc(base + (bpw - 2) * blk, blk, 0).wait()
                compute(0, tail // _L, tbase)
                tail_out.start()
                tail_out.wait()

            @pl.when(wid != last)
            def _():
                out_desc(base + (bpw - 2) * blk, blk, 0).wait()
        else:
            out_desc(base + (bpw - 2) * blk, blk, 0).wait()
        out_desc(base + (bpw - 1) * blk, blk, 1).wait()

    grid_kernel = pl.kernel(
        body,
        out_type=jax.ShapeDtypeStruct((4 * n,), jnp.int32),
        mesh=mesh,
        compiler_params=pltpu.CompilerParams(needs_layout_passes=False),
        scratch_types=[
            pltpu.VMEM((blk + h,), jnp.int32),
            pltpu.VMEM((blk + h,), jnp.int32),
            pltpu.VMEM((4 * blk,), jnp.int32),
            pltpu.VMEM((4 * blk,), jnp.int32),
            pltpu.VMEM((_L,), jnp.int32),
            pltpu.SemaphoreType.DMA,
            pltpu.SemaphoreType.DMA,
            pltpu.SemaphoreType.DMA,
            pltpu.SemaphoreType.DMA,
        ],
    )
    return grid_kernel


def kernel(paragraph_doc_ids, max_paragraphs):
    n = paragraph_doc_ids.shape[0]
    ids32 = paragraph_doc_ids.astype(jnp.int32)
    mp_vec = jnp.full((_L,), jnp.asarray(max_paragraphs, jnp.int32) - 1)
    out32 = _seg_pos_sc(n)(ids32, mp_vec)
    return out32.reshape(n, 4).astype(jnp.int64)


# speedup vs baseline: 14.9607x; 14.9607x over previous
import jax
import jax.numpy as jnp
from jax import lax
from jax.experimental import pallas as pl
from jax.experimental.pallas import tpu as pltpu
from jax.experimental.pallas import tpu_sc as plsc

_NC = 2
_NS = 16
_NW = _NC * _NS
_L = 16


def _seg_pos_sc(n):
    blk = 3840
    tpb = blk // 128
    nfull = n // blk
    rem = n - nfull * blk
    p = (n + 127) // 128
    h = _L
    assert rem % _L == 0 and (nfull + bool(rem)) <= 9 * _NW
    remw = nfull % _NW
    tot = -(-nfull // _NW)
    rtiles = -(-rem // 128)

    mesh = plsc.VectorSubcoreMesh(core_axis_name="c", subcore_axis_name="s",
                                  num_cores=_NC, num_subcores=_NS)

    def body(ids_hbm, mp_hbm, out_hbm, in_v0, in_v1, out_v0, out_v1, mp_v,
             in_sem0, in_sem1, out_sem0, out_sem1):
        in_bufs = (in_v0, in_v1)
        out_bufs = (out_v0, out_v1)
        in_sems = (in_sem0, in_sem1)
        out_sems = (out_sem0, out_sem1)
        wid = lax.axis_index("s") * _NC + lax.axis_index("c")
        nblk = jnp.where(wid < remw, tot, tot - 1)
        iota = lax.iota(jnp.int32, _L)

        pltpu.sync_copy(mp_hbm, mp_v)
        mp = mp_v[...]

        def bk(k):
            return wid + k * _NW

        def in_desc(k, b):
            return pltpu.make_async_copy(
                ids_hbm.at[pl.ds(bk(k) * blk - h, blk + h)],
                in_bufs[b], in_sems[b])

        def in_desc0(b):
            return pltpu.make_async_copy(
                ids_hbm.at[pl.ds(0, blk)],
                in_bufs[b].at[pl.ds(h, blk)], in_sems[b])

        def start_in(k, b, start=True):
            if k == 0:
                @pl.when(wid == 0)
                def _():
                    d = in_desc0(b)
                    d.start() if start else d.wait()

                @pl.when(wid > 0)
                def _():
                    d = in_desc(k, b)
                    d.start() if start else d.wait()
            else:
                d = in_desc(k, b)
                d.start() if start else d.wait()

        def out_desc(k, b):
            return pltpu.make_async_copy(
                out_bufs[b],
                out_hbm.at[pl.ds(bk(k) * tpb, tpb), :, :], out_sems[b])

        def compute(b, nv, gbase):
            inb = in_bufs[b]
            outb = out_bufs[b]

            @plsc.parallel_loop(0, nv, 1, unroll=4)
            def _(i):
                e = i * _L
                cur = inb[pl.ds(e + h, _L)]
                prev = plsc.load_gather(inb, [iota + (e + h - 1)])
                gi = iota + (gbase + e)
                d0 = jnp.where((cur != prev) | (gi == 0), 1, 0).astype(jnp.int32)
                d1 = jnp.where(cur == 0, 1, 0).astype(jnp.int32)
                d2 = jnp.where((cur > 0) & (cur < mp), 1, 0).astype(jnp.int32)
                d3 = jnp.where(cur == mp, 1, 0).astype(jnp.int32)
                t = i >> 3
                l0 = (i & 7) * _L
                outb[t, 0, pl.ds(l0, _L)] = d0
                outb[t, 1, pl.ds(l0, _L)] = d1
                outb[t, 2, pl.ds(l0, _L)] = d2
                outb[t, 3, pl.ds(l0, _L)] = d3

        if rem:
            rem_in = pltpu.make_async_copy(
                ids_hbm.at[pl.ds(nfull * blk - h, rem + h)],
                in_bufs[0].at[pl.ds(0, rem + h)], in_sems[0])
            rem_out = pltpu.make_async_copy(
                out_bufs[0].at[pl.ds(0, rtiles), :, :],
                out_hbm.at[pl.ds(nfull * tpb, rtiles), :, :], out_sems[0])

        start_in(0, 0)
        start_in(1, 1)
        for k in range(tot):
            b = k & 1

            @pl.when(k < nblk)
            def _(k=k, b=b):
                start_in(k, b, start=False)
                if k >= 2:
                    out_desc(k - 2, b).wait()
                compute(b, blk // _L, bk(k) * blk)
                out_desc(k, b).start()

                @pl.when(k + 2 < nblk)
                def _():
                    start_in(k + 2, b)

                if rem and k == tot - 3:
                    @pl.when(wid == remw)
                    def _():
                        rem_in.start()

        if rem:
            @pl.when(wid == remw)
            def _():
                rem_in.wait()
                out_desc(tot - 3, 0).wait()
                compute(0, rem // _L, nfull * blk)
                rem_out.start()
                rem_out.wait()

        @pl.when(wid < remw)
        def _():
            out_desc(tot - 1, (tot - 1) & 1).wait()

        @pl.when(wid > remw)
        def _():
            out_desc(tot - 3, (tot - 3) & 1).wait()
        out_desc(tot - 2, (tot - 2) & 1).wait()

    grid_kernel = pl.kernel(
        body,
        out_type=jax.ShapeDtypeStruct((p, 4, 128), jnp.int32),
        mesh=mesh,
        compiler_params=pltpu.CompilerParams(needs_layout_passes=False,
                                             use_tc_tiling_on_sc=False),
        scratch_types=[
            pltpu.VMEM((blk + h,), jnp.int32),
            pltpu.VMEM((blk + h,), jnp.int32),
            pltpu.VMEM((tpb, 4, 128), jnp.int32),
            pltpu.VMEM((tpb, 4, 128), jnp.int32),
            pltpu.VMEM((_L,), jnp.int32),
            pltpu.SemaphoreType.DMA,
            pltpu.SemaphoreType.DMA,
            pltpu.SemaphoreType.DMA,
            pltpu.SemaphoreType.DMA,
        ],
    )
    return grid_kernel


def kernel(paragraph_doc_ids, max_paragraphs):
    n = paragraph_doc_ids.shape[0]
    p = (n + 127) // 128
    ids32 = paragraph_doc_ids.astype(jnp.int32)
    mp_vec = jnp.full((_L,), jnp.asarray(max_paragraphs, jnp.int32) - 1)
    o3 = _seg_pos_sc(n)(ids32, mp_vec)
    vec = o3.transpose(0, 2, 1).reshape(p * 128, 4)[:n]
    return vec.astype(jnp.int64)
